# baseline (speedup 1.0000x reference)
import functools

import jax
import jax.numpy as jnp
from jax import lax
from jax.experimental import pallas as pl
from jax.experimental.pallas import tpu as pltpu
from jax.experimental.pallas import tpu_sc as plsc

N_NODES = 10000
N_EDGES = 320000
D = 128
N_GRAPHS = 64

NC = 2
NS = 16
NW = NC * NS
EPT = N_EDGES // NW
K = 40
NCHUNK = EPT // K
NPAD = 10240
RPT = NPAD // NS
LANES = 16
NR = 3
NI = 6
UNROLL = 6
NITER = (NCHUNK + NR) // UNROLL + 1

def _sc_agg_body(x_hbm, src_hbm, dst_hbm, parts_hbm, acc, *scratch):
  srcb = list(scratch[0:NI])
  dstb = list(scratch[NI:2 * NI])
  rows = list(scratch[2 * NI:2 * NI + NR])
  gsem = list(scratch[2 * NI + NR:2 * NI + 2 * NR])
  ssem = list(scratch[2 * NI + 2 * NR:2 * NI + 3 * NR])
  isem = list(scratch[2 * NI + 3 * NR:3 * NI + 3 * NR])

  cid = lax.axis_index("c")
  sid = lax.axis_index("s")
  wid = cid * NS + sid
  base = wid * EPT

  def idx_load(chunk_i, ring):
    off = pl.multiple_of(base + chunk_i * K, 8)
    pltpu.async_copy(src_hbm.at[pl.ds(off, K)], srcb[ring], isem[ring])
    pltpu.async_copy(dst_hbm.at[pl.ds(off, K)], dstb[ring], isem[ring])

  for c in range(NR):
    idx_load(c, c)

  zeros16 = jnp.zeros((LANES,), jnp.float32)

  def zbody(r, carry):
    for c in range(D // LANES):
      rows[0][r, pl.ds(c * LANES, LANES)] = zeros16
    return carry

  lax.fori_loop(0, K, zbody, 0)
  for k in range(RPT // K):
    pltpu.async_copy(rows[0], acc.at[pl.ds(sid * RPT + k * K, K)], gsem[0])
  for k in range(RPT // K):
    pltpu.make_async_copy(rows[0], acc.at[pl.ds(sid * RPT + k * K, K)],
                          gsem[0]).wait()
  plsc.subcore_barrier()

  def pipeline(g, carry):
    for b in range(UNROLL):
      i = g * UNROLL + b
      rb = b % NR

      @pl.when(jnp.logical_and(i >= NR, i < NCHUNK + NR))
      def _():
        pltpu.make_async_copy(rows[rb], acc.at[dstb[(b - NR) % NI]],
                              ssem[rb]).wait()

      @pl.when(i + NR < NCHUNK)
      def _():
        idx_load(i + NR, (b + NR) % NI)

      @pl.when(i < NCHUNK)
      def _():
        pltpu.make_async_copy(src_hbm.at[pl.ds(0, K)], srcb[b % NI],
                              isem[b % NI]).wait()
        pltpu.make_async_copy(dst_hbm.at[pl.ds(0, K)], dstb[b % NI],
                              isem[b % NI]).wait()
        pltpu.async_copy(x_hbm.at[srcb[b % NI]], rows[rb], gsem[rb])

      @pl.when(jnp.logical_and(i >= 1, i <= NCHUNK))
      def _():
        pltpu.make_async_copy(x_hbm.at[srcb[(b - 1) % NI]],
                              rows[(b - 1) % NR],
                              gsem[(b - 1) % NR]).wait()
        pltpu.async_copy(rows[(b - 1) % NR], acc.at[dstb[(b - 1) % NI]],
                         ssem[(b - 1) % NR], add=True)
    return carry

  lax.fori_loop(0, NITER, pipeline, 0)
  plsc.subcore_barrier()

  pltpu.sync_copy(acc.at[pl.ds(sid * RPT, RPT)],
                  parts_hbm.at[cid, pl.ds(sid * RPT, RPT)])


def _sc_agg(x, src, dst):
  mesh = plsc.VectorSubcoreMesh(
      core_axis_name="c", subcore_axis_name="s", num_cores=NC, num_subcores=NS
  )
  call = pl.kernel(
      _sc_agg_body,
      out_type=jax.ShapeDtypeStruct((NC, NPAD, D), jnp.float32),
      mesh=mesh,
      scratch_types=(
          [pltpu.VMEM_SHARED((NPAD, D), jnp.float32)]
          + [pltpu.VMEM((K,), jnp.int32) for _ in range(2 * NI)]
          + [pltpu.VMEM((K, D), jnp.float32) for _ in range(NR)]
          + [pltpu.SemaphoreType.DMA for _ in range(2 * NR + NI)]
      ),
      name="sc_edge_agg",
  )
  return call(x, src, dst)

_BLK = 1000
_NBLK = N_NODES // _BLK


def _pre_body(x_ref, wroot_ref, b_ref, batch_ref, z_ref, cnt_ref, cnt_acc):
  i = pl.program_id(0)

  @pl.when(i == 0)
  def _():
    cnt_acc[...] = jnp.zeros_like(cnt_acc)

  z = lax.dot_general(x_ref[...], wroot_ref[...], (((1,), (1,)), ((), ())),
                      preferred_element_type=jnp.float32)
  z_ref[...] = z + b_ref[...]

  b_ids = batch_ref[0, 0, :]
  onehot = (b_ids[:, None] ==
            lax.broadcasted_iota(jnp.int32, (1, N_GRAPHS), 1)
            ).astype(jnp.float32)
  cnt_acc[...] += jnp.sum(onehot, axis=0, keepdims=True)

  @pl.when(i == _NBLK - 1)
  def _():
    cnt_ref[...] = cnt_acc[...]


def _pre_stage(x, w_root, b_rel, batch3):
  return pl.pallas_call(
      _pre_body,
      grid=(_NBLK,),
      in_specs=[
          pl.BlockSpec((_BLK, D), lambda i: (i, 0)),
          pl.BlockSpec((D, D), lambda i: (0, 0)),
          pl.BlockSpec((1, D), lambda i: (0, 0)),
          pl.BlockSpec((1, 1, _BLK), lambda i: (i, 0, 0)),
      ],
      out_specs=[
          pl.BlockSpec((_BLK, D), lambda i: (i, 0)),
          pl.BlockSpec((1, N_GRAPHS), lambda i: (0, 0)),
      ],
      out_shape=[
          jax.ShapeDtypeStruct((N_NODES, D), jnp.float32),
          jax.ShapeDtypeStruct((1, N_GRAPHS), jnp.float32),
      ],
      scratch_shapes=[pltpu.VMEM((1, N_GRAPHS), jnp.float32)],
  )(x, w_root, b_rel.reshape(1, D), batch3)


def _lin_body(p_ref, z_ref, wrel_ref, o_ref):
  agg = p_ref[0] + p_ref[1]
  h = lax.dot_general(agg, wrel_ref[...], (((1,), (1,)), ((), ())),
                      preferred_element_type=jnp.float32)
  o_ref[...] = jnp.maximum(h + z_ref[...], 0.0)


def _layer_linear(parts, z1, w_rel):
  return pl.pallas_call(
      _lin_body,
      grid=(_NBLK,),
      in_specs=[
          pl.BlockSpec((NC, _BLK, D), lambda i: (0, i, 0)),
          pl.BlockSpec((_BLK, D), lambda i: (i, 0)),
          pl.BlockSpec((D, D), lambda i: (0, 0)),
      ],
      out_specs=pl.BlockSpec((_BLK, D), lambda i: (i, 0)),
      out_shape=jax.ShapeDtypeStruct((N_NODES, D), jnp.float32),
  )(parts, z1, w_rel)


def _final_body(p_ref, h1_ref, batch_ref, wrel_ref, wroot_ref, b_ref,
                wfc_ref, bfc_ref, cnt_ref, o_ref, pooled_acc):
  i = pl.program_id(0)

  @pl.when(i == 0)
  def _():
    pooled_acc[...] = jnp.zeros_like(pooled_acc)

  agg = p_ref[0] + p_ref[1]
  h = lax.dot_general(agg, wrel_ref[...], (((1,), (1,)), ((), ())),
                      preferred_element_type=jnp.float32)
  h += lax.dot_general(h1_ref[...], wroot_ref[...], (((1,), (1,)), ((), ())),
                       preferred_element_type=jnp.float32)
  h += b_ref[...]
  h = jnp.maximum(h, 0.0)

  b_ids = batch_ref[0, 0, :]
  onehot = (b_ids[:, None] ==
            lax.broadcasted_iota(jnp.int32, (1, N_GRAPHS), 1)
            ).astype(jnp.float32)
  pooled_acc[...] += lax.dot_general(
      onehot, h, (((0,), (0,)), ((), ())), preferred_element_type=jnp.float32)

  @pl.when(i == _NBLK - 1)
  def _():
    counts = jnp.clip(cnt_ref[...].reshape(N_GRAPHS, 1), 1.0, None)
    pooled = pooled_acc[...] / counts
    out = lax.dot_general(pooled, wfc_ref[...], (((1,), (1,)), ((), ())),
                          preferred_element_type=jnp.float32)
    o_ref[...] = out + bfc_ref[...]


def _final_stage(parts, h1, batch3, w_rel, b_rel, w_root, w_fc, b_fc, cnt):
  d_out = w_fc.shape[0]
  return pl.pallas_call(
      _final_body,
      grid=(_NBLK,),
      in_specs=[
          pl.BlockSpec((NC, _BLK, D), lambda i: (0, i, 0)),
          pl.BlockSpec((_BLK, D), lambda i: (i, 0)),
          pl.BlockSpec((1, 1, _BLK), lambda i: (i, 0, 0)),
          pl.BlockSpec((D, D), lambda i: (0, 0)),
          pl.BlockSpec((D, D), lambda i: (0, 0)),
          pl.BlockSpec((1, D), lambda i: (0, 0)),
          pl.BlockSpec((d_out, D), lambda i: (0, 0)),
          pl.BlockSpec((1, d_out), lambda i: (0, 0)),
          pl.BlockSpec((1, N_GRAPHS), lambda i: (0, 0)),
      ],
      out_specs=pl.BlockSpec((N_GRAPHS, d_out), lambda i: (0, 0)),
      out_shape=jax.ShapeDtypeStruct((N_GRAPHS, d_out), jnp.float32),
      scratch_shapes=[
          pltpu.VMEM((N_GRAPHS, D), jnp.float32),
      ],
  )(parts, h1, batch3, w_rel, w_root, b_rel.reshape(1, D),
    w_fc, b_fc.reshape(1, d_out), cnt)


@jax.jit
def kernel(x, edge_index, batch, W1_rel, b1_rel, W1_root,
           W2_rel, b2_rel, W2_root, W_fc, b_fc):
  src = edge_index[0]
  dst = edge_index[1]
  batch3 = batch.reshape(_NBLK, 1, _BLK)

  z1, cnt = _pre_stage(x, W1_root, b1_rel, batch3)
  parts1 = _sc_agg(x, src, dst)
  h1 = _layer_linear(parts1, z1, W1_rel)
  parts2 = _sc_agg(h1, src, dst)
  return _final_stage(parts2, h1, batch3, W2_rel, b2_rel, W2_root,
                      W_fc, b_fc, cnt)


# speedup vs baseline: 1.2701x; 1.2701x over previous
import functools

import jax
import jax.numpy as jnp
from jax import lax
from jax.experimental import pallas as pl
from jax.experimental.pallas import tpu as pltpu
from jax.experimental.pallas import tpu_sc as plsc

N_NODES = 10000
N_EDGES = 320000
D = 128
N_GRAPHS = 64

NC = 2
NS = 16
NW = NC * NS
EPT = N_EDGES // NW
K = 80
NCHUNK = EPT // K
NPAD = 10240
RPT = NPAD // NS
LANES = 16
NR = 3
NI = 6
UNROLL = 6
NITER = (NCHUNK + NR) // UNROLL + 1

ZB = 40

def _sc_agg_body(x_hbm, src_hbm, dst_hbm, parts_hbm, acc, *scratch):
  srcb = list(scratch[0:NI])
  dstb = list(scratch[NI:2 * NI])
  rows = list(scratch[2 * NI:2 * NI + NR])
  gsem = list(scratch[2 * NI + NR:2 * NI + 2 * NR])
  ssem = list(scratch[2 * NI + 2 * NR:2 * NI + 3 * NR])
  isem = list(scratch[2 * NI + 3 * NR:3 * NI + 3 * NR])
  zbuf = scratch[3 * NI + 3 * NR]
  zsem = scratch[3 * NI + 3 * NR + 1]

  cid = lax.axis_index("c")
  sid = lax.axis_index("s")
  wid = cid * NS + sid
  base = wid * EPT

  def idx_load(chunk_i, ring):
    off = pl.multiple_of(base + chunk_i * K, 8)
    pltpu.async_copy(src_hbm.at[pl.ds(off, K)], srcb[ring], isem[ring])
    pltpu.async_copy(dst_hbm.at[pl.ds(off, K)], dstb[ring], isem[ring])

  for c in range(NR):
    idx_load(c, c)
  for c in range(NR):
    pltpu.make_async_copy(src_hbm.at[pl.ds(0, K)], srcb[c], isem[c]).wait()
    pltpu.make_async_copy(dst_hbm.at[pl.ds(0, K)], dstb[c], isem[c]).wait()
    pltpu.async_copy(x_hbm.at[srcb[c]], rows[c], gsem[c])

  zeros16 = jnp.zeros((LANES,), jnp.float32)

  def zbody(r, carry):
    for c in range(D // LANES):
      zbuf[r, pl.ds(c * LANES, LANES)] = zeros16
    return carry

  lax.fori_loop(0, ZB, zbody, 0)
  for k in range(RPT // ZB):
    pltpu.async_copy(zbuf, acc.at[pl.ds(sid * RPT + k * ZB, ZB)], zsem)
  for k in range(RPT // ZB):
    pltpu.make_async_copy(zbuf, acc.at[pl.ds(sid * RPT + k * ZB, ZB)],
                          zsem).wait()
  plsc.subcore_barrier()

  def pipeline(g, carry):
    for b in range(UNROLL):
      i = g * UNROLL + b
      rb = b % NR

      @pl.when(jnp.logical_and(i >= NR, i < NCHUNK + NR))
      def _():
        pltpu.make_async_copy(rows[rb], acc.at[dstb[(b - NR) % NI]],
                              ssem[rb]).wait()

      @pl.when(i + NR < NCHUNK)
      def _():
        idx_load(i + NR, (b + NR) % NI)

      @pl.when(jnp.logical_and(i >= NR, i < NCHUNK))
      def _():
        pltpu.make_async_copy(src_hbm.at[pl.ds(0, K)], srcb[b % NI],
                              isem[b % NI]).wait()
        pltpu.make_async_copy(dst_hbm.at[pl.ds(0, K)], dstb[b % NI],
                              isem[b % NI]).wait()
        pltpu.async_copy(x_hbm.at[srcb[b % NI]], rows[rb], gsem[rb])

      @pl.when(jnp.logical_and(i >= 1, i <= NCHUNK))
      def _():
        pltpu.make_async_copy(x_hbm.at[srcb[(b - 1) % NI]],
                              rows[(b - 1) % NR],
                              gsem[(b - 1) % NR]).wait()
        pltpu.async_copy(rows[(b - 1) % NR], acc.at[dstb[(b - 1) % NI]],
                         ssem[(b - 1) % NR], add=True)
    return carry

  lax.fori_loop(0, NITER, pipeline, 0)
  plsc.subcore_barrier()

  pltpu.sync_copy(acc.at[pl.ds(sid * RPT, RPT)],
                  parts_hbm.at[cid, pl.ds(sid * RPT, RPT)])


def _sc_agg(x, src, dst):
  mesh = plsc.VectorSubcoreMesh(
      core_axis_name="c", subcore_axis_name="s", num_cores=NC, num_subcores=NS
  )
  call = pl.kernel(
      _sc_agg_body,
      out_type=jax.ShapeDtypeStruct((NC, NPAD, D), jnp.float32),
      mesh=mesh,
      scratch_types=(
          [pltpu.VMEM_SHARED((NPAD, D), jnp.float32)]
          + [pltpu.VMEM((K,), jnp.int32) for _ in range(2 * NI)]
          + [pltpu.VMEM((K, D), jnp.float32) for _ in range(NR)]
          + [pltpu.SemaphoreType.DMA for _ in range(2 * NR + NI)]
          + [pltpu.VMEM((ZB, D), jnp.float32), pltpu.SemaphoreType.DMA]
      ),
      name="sc_edge_agg",
  )
  return call(x, src, dst)

_BLK = 1000
_NBLK = N_NODES // _BLK


def _pre_body(x_ref, wroot_ref, b_ref, batch_ref, z_ref, cnt_ref, cnt_acc):
  i = pl.program_id(0)

  @pl.when(i == 0)
  def _():
    cnt_acc[...] = jnp.zeros_like(cnt_acc)

  z = lax.dot_general(x_ref[...], wroot_ref[...], (((1,), (1,)), ((), ())),
                      preferred_element_type=jnp.float32)
  z_ref[...] = z + b_ref[...]

  b_ids = batch_ref[0, 0, :]
  onehot = (b_ids[:, None] ==
            lax.broadcasted_iota(jnp.int32, (1, N_GRAPHS), 1)
            ).astype(jnp.float32)
  cnt_acc[...] += jnp.sum(onehot, axis=0, keepdims=True)

  @pl.when(i == _NBLK - 1)
  def _():
    cnt_ref[...] = cnt_acc[...]


def _pre_stage(x, w_root, b_rel, batch3):
  return pl.pallas_call(
      _pre_body,
      grid=(_NBLK,),
      in_specs=[
          pl.BlockSpec((_BLK, D), lambda i: (i, 0)),
          pl.BlockSpec((D, D), lambda i: (0, 0)),
          pl.BlockSpec((1, D), lambda i: (0, 0)),
          pl.BlockSpec((1, 1, _BLK), lambda i: (i, 0, 0)),
      ],
      out_specs=[
          pl.BlockSpec((_BLK, D), lambda i: (i, 0)),
          pl.BlockSpec((1, N_GRAPHS), lambda i: (0, 0)),
      ],
      out_shape=[
          jax.ShapeDtypeStruct((N_NODES, D), jnp.float32),
          jax.ShapeDtypeStruct((1, N_GRAPHS), jnp.float32),
      ],
      scratch_shapes=[pltpu.VMEM((1, N_GRAPHS), jnp.float32)],
  )(x, w_root, b_rel.reshape(1, D), batch3)


def _lin_body(p_ref, z_ref, wrel_ref, o_ref):
  agg = p_ref[0] + p_ref[1]
  h = lax.dot_general(agg, wrel_ref[...], (((1,), (1,)), ((), ())),
                      preferred_element_type=jnp.float32)
  o_ref[...] = jnp.maximum(h + z_ref[...], 0.0)


def _layer_linear(parts, z1, w_rel):
  return pl.pallas_call(
      _lin_body,
      grid=(_NBLK,),
      in_specs=[
          pl.BlockSpec((NC, _BLK, D), lambda i: (0, i, 0)),
          pl.BlockSpec((_BLK, D), lambda i: (i, 0)),
          pl.BlockSpec((D, D), lambda i: (0, 0)),
      ],
      out_specs=pl.BlockSpec((_BLK, D), lambda i: (i, 0)),
      out_shape=jax.ShapeDtypeStruct((N_NODES, D), jnp.float32),
  )(parts, z1, w_rel)


def _final_body(p_ref, h1_ref, batch_ref, wrel_ref, wroot_ref, b_ref,
                wfc_ref, bfc_ref, cnt_ref, o_ref, pooled_acc):
  i = pl.program_id(0)

  @pl.when(i == 0)
  def _():
    pooled_acc[...] = jnp.zeros_like(pooled_acc)

  agg = p_ref[0] + p_ref[1]
  h = lax.dot_general(agg, wrel_ref[...], (((1,), (1,)), ((), ())),
                      preferred_element_type=jnp.float32)
  h += lax.dot_general(h1_ref[...], wroot_ref[...], (((1,), (1,)), ((), ())),
                       preferred_element_type=jnp.float32)
  h += b_ref[...]
  h = jnp.maximum(h, 0.0)

  b_ids = batch_ref[0, 0, :]
  onehot = (b_ids[:, None] ==
            lax.broadcasted_iota(jnp.int32, (1, N_GRAPHS), 1)
            ).astype(jnp.float32)
  pooled_acc[...] += lax.dot_general(
      onehot, h, (((0,), (0,)), ((), ())), preferred_element_type=jnp.float32)

  @pl.when(i == _NBLK - 1)
  def _():
    counts = jnp.clip(cnt_ref[...].reshape(N_GRAPHS, 1), 1.0, None)
    pooled = pooled_acc[...] / counts
    out = lax.dot_general(pooled, wfc_ref[...], (((1,), (1,)), ((), ())),
                          preferred_element_type=jnp.float32)
    o_ref[...] = out + bfc_ref[...]


def _final_stage(parts, h1, batch3, w_rel, b_rel, w_root, w_fc, b_fc, cnt):
  d_out = w_fc.shape[0]
  return pl.pallas_call(
      _final_body,
      grid=(_NBLK,),
      in_specs=[
          pl.BlockSpec((NC, _BLK, D), lambda i: (0, i, 0)),
          pl.BlockSpec((_BLK, D), lambda i: (i, 0)),
          pl.BlockSpec((1, 1, _BLK), lambda i: (i, 0, 0)),
          pl.BlockSpec((D, D), lambda i: (0, 0)),
          pl.BlockSpec((D, D), lambda i: (0, 0)),
          pl.BlockSpec((1, D), lambda i: (0, 0)),
          pl.BlockSpec((d_out, D), lambda i: (0, 0)),
          pl.BlockSpec((1, d_out), lambda i: (0, 0)),
          pl.BlockSpec((1, N_GRAPHS), lambda i: (0, 0)),
      ],
      out_specs=pl.BlockSpec((N_GRAPHS, d_out), lambda i: (0, 0)),
      out_shape=jax.ShapeDtypeStruct((N_GRAPHS, d_out), jnp.float32),
      scratch_shapes=[
          pltpu.VMEM((N_GRAPHS, D), jnp.float32),
      ],
  )(parts, h1, batch3, w_rel, w_root, b_rel.reshape(1, D),
    w_fc, b_fc.reshape(1, d_out), cnt)


@jax.jit
def kernel(x, edge_index, batch, W1_rel, b1_rel, W1_root,
           W2_rel, b2_rel, W2_root, W_fc, b_fc):
  src = edge_index[0]
  dst = edge_index[1]
  batch3 = batch.reshape(_NBLK, 1, _BLK)

  z1, cnt = _pre_stage(x, W1_root, b1_rel, batch3)
  parts1 = _sc_agg(x, src, dst)
  h1 = _layer_linear(parts1, z1, W1_rel)
  parts2 = _sc_agg(h1, src, dst)
  return _final_stage(parts2, h1, batch3, W2_rel, b2_rel, W2_root,
                      W_fc, b_fc, cnt)
